# Initial kernel scaffold; baseline (speedup 1.0000x reference)
import jax
import jax.numpy as jnp
from jax.experimental import pallas as pl


def kernel(x, edge_index, W_self1, W_neigh1, b1, W_self2, W_neigh2, b2):
    raise NotImplementedError("write your pallas kernel here")


# speedup vs baseline: 4.1114x; 4.1114x over previous
import functools

import jax
import jax.numpy as jnp
from jax import lax
from jax.experimental import pallas as pl
from jax.experimental.pallas import tpu as pltpu
from jax.experimental.pallas import tpu_sc as plsc

_N = 10000
_E = 320000
_D = 128
_NC = 2
_NS = 16
_EPC = _E // _NC
_EPT = _E // (_NC * _NS)
_NP = 10240
_RPT = _NP // _NS
_CHUNK = 80

_vector_mesh = plsc.VectorSubcoreMesh(core_axis_name="c", subcore_axis_name="s")


@functools.partial(
    pl.kernel, mesh=_vector_mesh,
    out_type=jax.ShapeDtypeStruct((_NC * _NP, _D), jnp.float32),
    scratch_types=[
        pltpu.VMEM((_CHUNK,), jnp.int32),
        pltpu.VMEM((1, _CHUNK), jnp.int32),
        pltpu.VMEM((_CHUNK, _D), jnp.float32),
        pltpu.VMEM_SHARED((_NP, _D), jnp.float32),
        pltpu.SemaphoreType.DMA,
    ])
def _sc_layer(h_hbm, src_hbm, dst_hbm, out_hbm, src_v, dst_v, rows_v, acc_sh,
              sem):
  core = lax.axis_index("c")
  sub = lax.axis_index("s")

  rbase = sub * _RPT
  nchunks = _RPT // _CHUNK

  @pl.loop(0, _CHUNK)
  def _(i):
    @pl.loop(0, _D, step=16)
    def _(j):
      rows_v[i, pl.ds(j, 16)] = jnp.zeros((16,), jnp.float32)
  for j in range(nchunks):
    pltpu.sync_copy(rows_v, acc_sh.at[pl.ds(rbase + j * _CHUNK, _CHUNK)])
  plsc.subcore_barrier()

  ebase = core * _EPC + sub * _EPT

  @pl.loop(0, _EPT, step=_CHUNK)
  def _(e):
    off = ebase + e
    pltpu.sync_copy(src_hbm.at[pl.ds(off, _CHUNK)], src_v)
    pltpu.sync_copy(dst_hbm.at[pl.ds(off, _CHUNK)], dst_v.at[0])
    pltpu.async_copy(h_hbm.at[src_v], rows_v, sem).wait()
    pltpu.sync_copy(rows_v, acc_sh.at[dst_v.at[0]], add=True)
  plsc.subcore_barrier()

  obase = core * _NP + rbase
  for j in range(nchunks):
    pltpu.sync_copy(acc_sh.at[pl.ds(rbase + j * _CHUNK, _CHUNK)], rows_v)
    pltpu.sync_copy(rows_v, out_hbm.at[pl.ds(obase + j * _CHUNK, _CHUNK)])


@functools.partial(
    pl.kernel, mesh=_vector_mesh,
    out_type=jax.ShapeDtypeStruct((_NC * _NP, _D), jnp.float32),
    scratch_types=[
        pltpu.VMEM((1, _CHUNK), jnp.int32),
        pltpu.VMEM((_CHUNK, _D), jnp.float32),
        pltpu.VMEM_SHARED((_NP, _D), jnp.float32),
    ])
def _sc_deg(dst_hbm, out_hbm, dst_v, ones_v, acc_sh):
  core = lax.axis_index("c")
  sub = lax.axis_index("s")

  rbase = sub * _RPT
  nchunks = _RPT // _CHUNK

  @pl.loop(0, _CHUNK)
  def _(i):
    @pl.loop(0, _D, step=16)
    def _(j):
      ones_v[i, pl.ds(j, 16)] = jnp.zeros((16,), jnp.float32)
  for j in range(nchunks):
    pltpu.sync_copy(ones_v, acc_sh.at[pl.ds(rbase + j * _CHUNK, _CHUNK)])

  @pl.loop(0, _CHUNK)
  def _(i):
    @pl.loop(0, _D, step=16)
    def _(j):
      ones_v[i, pl.ds(j, 16)] = jnp.ones((16,), jnp.float32)
  plsc.subcore_barrier()

  ebase = core * _EPC + sub * _EPT

  @pl.loop(0, _EPT, step=_CHUNK)
  def _(e):
    pltpu.sync_copy(dst_hbm.at[pl.ds(ebase + e, _CHUNK)], dst_v.at[0])
    pltpu.sync_copy(ones_v, acc_sh.at[dst_v.at[0]], add=True)
  plsc.subcore_barrier()

  obase = core * _NP + rbase
  for j in range(nchunks):
    pltpu.sync_copy(acc_sh.at[pl.ds(rbase + j * _CHUNK, _CHUNK)], ones_v)
    pltpu.sync_copy(ones_v, out_hbm.at[pl.ds(obase + j * _CHUNK, _CHUNK)])


_RB = 80


def _dense_update(x, acc, deg, W_self, W_neigh, b, apply_relu):
  nb = _N // _RB

  def body(x_ref, a0, a1, d0, d1, ws, wn, b_ref, o_ref):
    s = a0[...] + a1[...]
    dcol = d0[...][:, 0:1] + d1[...][:, 0:1]
    hn = s / jnp.maximum(dcol, 1.0)
    out = (jnp.dot(x_ref[...], ws[...], preferred_element_type=jnp.float32,
                   precision=lax.Precision.HIGHEST)
           + jnp.dot(hn, wn[...], preferred_element_type=jnp.float32,
                     precision=lax.Precision.HIGHEST)
           + b_ref[...])
    if apply_relu:
      out = jnp.maximum(out, 0.0)
    o_ref[...] = out

  return pl.pallas_call(
      body,
      grid=(nb,),
      in_specs=[
          pl.BlockSpec((_RB, _D), lambda i: (i, 0)),
          pl.BlockSpec((_RB, _D), lambda i: (i, 0)),
          pl.BlockSpec((_RB, _D), lambda i: (i + _NP // _RB, 0)),
          pl.BlockSpec((_RB, _D), lambda i: (i, 0)),
          pl.BlockSpec((_RB, _D), lambda i: (i + _NP // _RB, 0)),
          pl.BlockSpec((_D, _D), lambda i: (0, 0)),
          pl.BlockSpec((_D, _D), lambda i: (0, 0)),
          pl.BlockSpec((1, _D), lambda i: (0, 0)),
      ],
      out_specs=pl.BlockSpec((_RB, _D), lambda i: (i, 0)),
      out_shape=jax.ShapeDtypeStruct((_N, _D), jnp.float32),
  )(x, acc, acc, deg, deg, W_self, W_neigh, b.reshape(1, _D))


def kernel(x, edge_index, W_self1, W_neigh1, b1, W_self2, W_neigh2, b2):
  src = edge_index[0]
  dst = edge_index[1]
  deg = _sc_deg(dst)
  acc1 = _sc_layer(x, src, dst)
  h1 = _dense_update(x, acc1, deg, W_self1, W_neigh1, b1, True)
  acc2 = _sc_layer(h1, src, dst)
  h2 = _dense_update(h1, acc2, deg, W_self2, W_neigh2, b2, False)
  return h2


# speedup vs baseline: 6.1108x; 1.4863x over previous
import functools

import jax
import jax.numpy as jnp
from jax import lax
from jax.experimental import pallas as pl
from jax.experimental.pallas import tpu as pltpu
from jax.experimental.pallas import tpu_sc as plsc

_N = 10000
_E = 320000
_D = 128
_NC = 2
_NS = 16
_NW = _NC * _NS
_NP = 10240
_RPT = _NP // _NS
_CHUNK = 80
_GC = 8
_NGRP = _E // (_CHUNK * _GC)
_GBASE = _NGRP // _NW
_GEXTRA = _NGRP - _GBASE * _NW

_vector_mesh = plsc.VectorSubcoreMesh(core_axis_name="c", subcore_axis_name="s")


def _fill_rows(buf, value):
  @pl.loop(0, _CHUNK)
  def _(i):
    @pl.loop(0, _D, step=16)
    def _(j):
      buf[i, pl.ds(j, 16)] = jnp.full((16,), value, jnp.float32)


def _tile_groups(core, sub):
  t = core * _NS + sub
  start = t * _GBASE + jnp.minimum(t, _GEXTRA)
  ngroups = _GBASE + jnp.where(t < _GEXTRA, 1, 0)
  return start, ngroups


@functools.partial(
    pl.kernel, mesh=_vector_mesh,
    out_type=jax.ShapeDtypeStruct((_NC * _NP, _D), jnp.float32),
    scratch_types=[
        pltpu.VMEM((2, _GC, _CHUNK), jnp.int32),
        pltpu.VMEM((2, _GC, _CHUNK), jnp.int32),
        pltpu.VMEM((_CHUNK, _D), jnp.float32),
        pltpu.VMEM((_CHUNK, _D), jnp.float32),
        pltpu.VMEM_SHARED((_NP, _D), jnp.float32),
        pltpu.SemaphoreType.DMA,
        pltpu.SemaphoreType.DMA,
        pltpu.SemaphoreType.DMA,
    ])
def _sc_layer(h_hbm, src_hbm, dst_hbm, out_hbm, src_blk, dst_blk,
              rows0, rows1, acc_sh, gsem, ssem0, ssem1):
  core = lax.axis_index("c")
  sub = lax.axis_index("s")

  rbase = sub * _RPT
  nchunks = _RPT // _CHUNK
  rows = (rows0, rows1)
  ssem = (ssem0, ssem1)

  _fill_rows(rows0, 0.0)
  _fill_rows(rows1, 0.0)
  for j in range(nchunks):
    pltpu.sync_copy(rows0, acc_sh.at[pl.ds(rbase + j * _CHUNK, _CHUNK)])
  plsc.subcore_barrier()

  gstart, ngroups = _tile_groups(core, sub)

  def drain_scatter(b):
    pltpu.make_async_copy(h_hbm.at[pl.ds(0, _CHUNK)], rows[b], ssem[b]).wait()

  @pl.loop(0, _CHUNK, step=16)
  def _(i):
    dst_blk[0, 0, pl.ds(i, 16)] = jnp.full((16,), 0, jnp.int32) + rbase
  pltpu.async_copy(rows0, acc_sh.at[dst_blk.at[0].at[0]], ssem0, add=True)
  pltpu.async_copy(rows1, acc_sh.at[dst_blk.at[0].at[0]], ssem1, add=True)

  @pl.loop(0, ngroups)
  def _(blk):
    pb = blk % 2
    roff = (gstart + blk) * _GC
    pltpu.sync_copy(src_hbm.at[pl.ds(roff, _GC)], src_blk.at[pb])
    pltpu.sync_copy(dst_hbm.at[pl.ds(roff, _GC)], dst_blk.at[pb])
    for c in range(_GC):
      b = c % 2
      drain_scatter(b)
      pltpu.async_copy(h_hbm.at[src_blk.at[pb].at[c]], rows[b], gsem).wait()
      pltpu.async_copy(rows[b], acc_sh.at[dst_blk.at[pb].at[c]], ssem[b],
                       add=True)

  drain_scatter(0)
  drain_scatter(1)
  plsc.subcore_barrier()

  obase = core * _NP + rbase
  for j in range(nchunks):
    pltpu.sync_copy(acc_sh.at[pl.ds(rbase + j * _CHUNK, _CHUNK)], rows0)
    pltpu.sync_copy(rows0, out_hbm.at[pl.ds(obase + j * _CHUNK, _CHUNK)])


@functools.partial(
    pl.kernel, mesh=_vector_mesh,
    out_type=jax.ShapeDtypeStruct((_NC * _NP, _D), jnp.float32),
    scratch_types=[
        pltpu.VMEM((2, _GC, _CHUNK), jnp.int32),
        pltpu.VMEM((_CHUNK, _D), jnp.float32),
        pltpu.VMEM_SHARED((_NP, _D), jnp.float32),
        pltpu.SemaphoreType.DMA,
    ])
def _sc_deg(dst_hbm, out_hbm, dst_blk, ones_v, acc_sh, ssem):
  core = lax.axis_index("c")
  sub = lax.axis_index("s")

  rbase = sub * _RPT
  nchunks = _RPT // _CHUNK

  _fill_rows(ones_v, 0.0)
  for j in range(nchunks):
    pltpu.sync_copy(ones_v, acc_sh.at[pl.ds(rbase + j * _CHUNK, _CHUNK)])
  _fill_rows(ones_v, 1.0)
  plsc.subcore_barrier()

  gstart, ngroups = _tile_groups(core, sub)

  def drain_scatter():
    pltpu.make_async_copy(out_hbm.at[pl.ds(0, _CHUNK)], ones_v, ssem).wait()

  @pl.loop(0, ngroups)
  def _(blk):
    pb = blk % 2
    roff = (gstart + blk) * _GC
    pltpu.sync_copy(dst_hbm.at[pl.ds(roff, _GC)], dst_blk.at[pb])
    for c in range(0, _GC, 2):
      pltpu.async_copy(ones_v, acc_sh.at[dst_blk.at[pb].at[c]], ssem,
                       add=True)
      pltpu.async_copy(ones_v, acc_sh.at[dst_blk.at[pb].at[c + 1]], ssem,
                       add=True)
      drain_scatter()
      drain_scatter()

  plsc.subcore_barrier()

  obase = core * _NP + rbase
  for j in range(nchunks):
    pltpu.sync_copy(acc_sh.at[pl.ds(rbase + j * _CHUNK, _CHUNK)], ones_v)
    pltpu.sync_copy(ones_v, out_hbm.at[pl.ds(obase + j * _CHUNK, _CHUNK)])


_RB = 80


def _dense_update(x, acc, deg, W_self, W_neigh, b, apply_relu):
  nb = _N // _RB

  def body(x_ref, a0, a1, d0, d1, ws, wn, b_ref, o_ref):
    s = a0[...] + a1[...]
    dcol = d0[...][:, 0:1] + d1[...][:, 0:1]
    hn = s / jnp.maximum(dcol, 1.0)
    out = (jnp.dot(x_ref[...], ws[...], preferred_element_type=jnp.float32,
                   precision=lax.Precision.HIGHEST)
           + jnp.dot(hn, wn[...], preferred_element_type=jnp.float32,
                     precision=lax.Precision.HIGHEST)
           + b_ref[...])
    if apply_relu:
      out = jnp.maximum(out, 0.0)
    o_ref[...] = out

  return pl.pallas_call(
      body,
      grid=(nb,),
      in_specs=[
          pl.BlockSpec((_RB, _D), lambda i: (i, 0)),
          pl.BlockSpec((_RB, _D), lambda i: (i, 0)),
          pl.BlockSpec((_RB, _D), lambda i: (i + _NP // _RB, 0)),
          pl.BlockSpec((_RB, _D), lambda i: (i, 0)),
          pl.BlockSpec((_RB, _D), lambda i: (i + _NP // _RB, 0)),
          pl.BlockSpec((_D, _D), lambda i: (0, 0)),
          pl.BlockSpec((_D, _D), lambda i: (0, 0)),
          pl.BlockSpec((1, _D), lambda i: (0, 0)),
      ],
      out_specs=pl.BlockSpec((_RB, _D), lambda i: (i, 0)),
      out_shape=jax.ShapeDtypeStruct((_N, _D), jnp.float32),
  )(x, acc, acc, deg, deg, W_self, W_neigh, b.reshape(1, _D))


def kernel(x, edge_index, W_self1, W_neigh1, b1, W_self2, W_neigh2, b2):
  src2 = edge_index[0].reshape(_E // _CHUNK, _CHUNK)
  dst2 = edge_index[1].reshape(_E // _CHUNK, _CHUNK)
  deg = _sc_deg(dst2)
  acc1 = _sc_layer(x, src2, dst2)
  h1 = _dense_update(x, acc1, deg, W_self1, W_neigh1, b1, True)
  acc2 = _sc_layer(h1, src2, dst2)
  h2 = _dense_update(h1, acc2, deg, W_self2, W_neigh2, b2, False)
  return h2


# speedup vs baseline: 7.4687x; 1.2222x over previous
import functools

import jax
import jax.numpy as jnp
from jax import lax
from jax.experimental import pallas as pl
from jax.experimental.pallas import tpu as pltpu
from jax.experimental.pallas import tpu_sc as plsc

_N = 10000
_E = 320000
_D = 128
_NC = 2
_NS = 16
_NW = _NC * _NS
_NP = 10240
_RPT = _NP // _NS
_CHUNK = 80
_GC = 8
_NGRP = _E // (_CHUNK * _GC)
_GBASE = _NGRP // _NW
_GEXTRA = _NGRP - _GBASE * _NW

_vector_mesh = plsc.VectorSubcoreMesh(core_axis_name="c", subcore_axis_name="s")


def _fill_rows(buf, value):
  @pl.loop(0, _CHUNK)
  def _(i):
    @pl.loop(0, _D, step=16)
    def _(j):
      buf[i, pl.ds(j, 16)] = jnp.full((16,), value, jnp.float32)


def _tile_groups(core, sub):
  t = core * _NS + sub
  start = t * _GBASE + jnp.minimum(t, _GEXTRA)
  ngroups = _GBASE + jnp.where(t < _GEXTRA, 1, 0)
  return start, ngroups


def _make_sc_layer(compute_deg):
  out_type = [jax.ShapeDtypeStruct((_NC * _NP, _D), jnp.float32)]
  if compute_deg:
    out_type.append(jax.ShapeDtypeStruct((_NC * _NP, _D), jnp.float32))

  @functools.partial(
      pl.kernel, mesh=_vector_mesh, out_type=out_type,
      scratch_types=[
          pltpu.VMEM((2, _GC, _CHUNK), jnp.int32),
          pltpu.VMEM((2, _GC, _CHUNK), jnp.int32),
          pltpu.VMEM((_CHUNK, _D), jnp.float32),
          pltpu.VMEM((_CHUNK, _D), jnp.float32),
          pltpu.VMEM_SHARED((_NP, _D), jnp.float32),
          pltpu.SemaphoreType.DMA,
          pltpu.SemaphoreType.DMA,
          pltpu.SemaphoreType.DMA,
      ])
  def k(h_hbm, src_hbm, dst_hbm, *refs):
    if compute_deg:
      (out_hbm, deg_hbm, src_blk, dst_blk, rows0, rows1, acc_sh,
       gsem, ssem0, ssem1) = refs
    else:
      (out_hbm, src_blk, dst_blk, rows0, rows1, acc_sh,
       gsem, ssem0, ssem1) = refs
    core = lax.axis_index("c")
    sub = lax.axis_index("s")

    rbase = sub * _RPT
    nchunks = _RPT // _CHUNK
    rows = (rows0, rows1)
    ssem = (ssem0, ssem1)

    def drain(sem):
      pltpu.make_async_copy(h_hbm.at[pl.ds(0, _CHUNK)], rows0, sem).wait()

    def zero_acc():
      for j in range(nchunks):
        pltpu.sync_copy(rows0, acc_sh.at[pl.ds(rbase + j * _CHUNK, _CHUNK)])

    def writeout(dst, bounce_pair):
      for j in range(nchunks):
        b = j % 2
        if j >= 2:
          drain(ssem[b])
        pltpu.sync_copy(acc_sh.at[pl.ds(rbase + j * _CHUNK, _CHUNK)],
                        bounce_pair[b])
        pltpu.async_copy(bounce_pair[b],
                         dst.at[pl.ds(core * _NP + rbase + j * _CHUNK,
                                      _CHUNK)],
                         ssem[b])
      drain(ssem[0])
      drain(ssem[1])

    gstart, ngroups = _tile_groups(core, sub)

    _fill_rows(rows0, 0.0)
    zero_acc()
    plsc.subcore_barrier()

    if compute_deg:
      _fill_rows(rows1, 1.0)

      @pl.loop(0, ngroups)
      def _(blk):
        pb = blk % 2
        roff = (gstart + blk) * _GC
        pltpu.sync_copy(dst_hbm.at[pl.ds(roff, _GC)], dst_blk.at[pb])
        for c in range(0, _GC, 2):
          pltpu.async_copy(rows1, acc_sh.at[dst_blk.at[pb].at[c]], ssem0,
                           add=True)
          pltpu.async_copy(rows1, acc_sh.at[dst_blk.at[pb].at[c + 1]], ssem0,
                           add=True)
          drain(ssem0)
          drain(ssem0)
      plsc.subcore_barrier()
      writeout(deg_hbm, (rows0, rows1))
      _fill_rows(rows0, 0.0)
      _fill_rows(rows1, 0.0)
      zero_acc()
      plsc.subcore_barrier()
    else:
      _fill_rows(rows1, 0.0)

    @pl.loop(0, _CHUNK, step=16)
    def _(i):
      dst_blk[0, 0, pl.ds(i, 16)] = jnp.full((16,), 0, jnp.int32) + rbase
    pltpu.async_copy(rows0, acc_sh.at[dst_blk.at[0].at[0]], ssem0, add=True)
    pltpu.async_copy(rows1, acc_sh.at[dst_blk.at[0].at[0]], ssem1, add=True)

    @pl.loop(0, ngroups)
    def _(blk):
      pb = blk % 2
      roff = (gstart + blk) * _GC
      pltpu.sync_copy(src_hbm.at[pl.ds(roff, _GC)], src_blk.at[pb])
      pltpu.sync_copy(dst_hbm.at[pl.ds(roff, _GC)], dst_blk.at[pb])
      for c in range(_GC):
        b = c % 2
        drain(ssem[b])
        pltpu.async_copy(h_hbm.at[src_blk.at[pb].at[c]], rows[b], gsem).wait()
        pltpu.async_copy(rows[b], acc_sh.at[dst_blk.at[pb].at[c]], ssem[b],
                         add=True)

    drain(ssem0)
    drain(ssem1)
    plsc.subcore_barrier()

    writeout(out_hbm, (rows0, rows1))

  return k


_sc_layer_deg = _make_sc_layer(True)
_sc_layer = _make_sc_layer(False)

_RB = 1000


def _dense_update(x, a0, a1, d0, d1, W_self, W_neigh, b, apply_relu):
  nb = _N // _RB

  def body(x_ref, a0_ref, a1_ref, d0_ref, d1_ref, ws, wn, b_ref, o_ref):
    s = a0_ref[...] + a1_ref[...]
    dcol = d0_ref[...][:, 0:1] + d1_ref[...][:, 0:1]
    hn = s / jnp.maximum(dcol, 1.0)
    out = (jnp.dot(x_ref[...], ws[...], preferred_element_type=jnp.float32,
                   precision=lax.Precision.HIGHEST)
           + jnp.dot(hn, wn[...], preferred_element_type=jnp.float32,
                     precision=lax.Precision.HIGHEST)
           + b_ref[...])
    if apply_relu:
      out = jnp.maximum(out, 0.0)
    o_ref[...] = out

  blk = lambda r: pl.BlockSpec((r, _D), lambda i: (i, 0))
  return pl.pallas_call(
      body,
      grid=(nb,),
      in_specs=[
          blk(_RB), blk(_RB), blk(_RB), blk(_RB), blk(_RB),
          pl.BlockSpec((_D, _D), lambda i: (0, 0)),
          pl.BlockSpec((_D, _D), lambda i: (0, 0)),
          pl.BlockSpec((1, _D), lambda i: (0, 0)),
      ],
      out_specs=blk(_RB),
      out_shape=jax.ShapeDtypeStruct((_N, _D), jnp.float32),
  )(x, a0, a1, d0, d1, W_self, W_neigh, b.reshape(1, _D))


def _halves(arr):
  return arr[:_N], arr[_NP:_NP + _N]


def kernel(x, edge_index, W_self1, W_neigh1, b1, W_self2, W_neigh2, b2):
  src2 = edge_index[0].reshape(_E // _CHUNK, _CHUNK)
  dst2 = edge_index[1].reshape(_E // _CHUNK, _CHUNK)
  acc1, deg = _sc_layer_deg(x, src2, dst2)
  a0, a1 = _halves(acc1)
  d0, d1 = _halves(deg)
  h1 = _dense_update(x, a0, a1, d0, d1, W_self1, W_neigh1, b1, True)
  (acc2,) = _sc_layer(h1, src2, dst2)
  b0, b1_ = _halves(acc2)
  h2 = _dense_update(h1, b0, b1_, d0, d1, W_self2, W_neigh2, b2, False)
  return h2


# speedup vs baseline: 8.0906x; 1.0833x over previous
import functools

import jax
import jax.numpy as jnp
from jax import lax
from jax.experimental import pallas as pl
from jax.experimental.pallas import tpu as pltpu
from jax.experimental.pallas import tpu_sc as plsc

_N = 10000
_E = 320000
_D = 128
_NC = 2
_NS = 16
_NW = _NC * _NS
_NP = 10240
_RPT = _NP // _NS
_CHUNK = 80
_GC = 8
_NGRP = _E // (_CHUNK * _GC)
_GBASE = _NGRP // _NW
_GEXTRA = _NGRP - _GBASE * _NW

_vector_mesh = plsc.VectorSubcoreMesh(core_axis_name="c", subcore_axis_name="s")


def _fill_rows(buf, value):
  @pl.loop(0, _CHUNK)
  def _(i):
    @pl.loop(0, _D, step=16)
    def _(j):
      buf[i, pl.ds(j, 16)] = jnp.full((16,), value, jnp.float32)


def _tile_groups(core, sub):
  t = core * _NS + sub
  start = t * _GBASE + jnp.minimum(t, _GEXTRA)
  ngroups = _GBASE + jnp.where(t < _GEXTRA, 1, 0)
  return start, ngroups


def _make_sc_layer(compute_deg):
  out_type = [jax.ShapeDtypeStruct((_NC * _NP, _D), jnp.float32)]
  if compute_deg:
    out_type.append(jax.ShapeDtypeStruct((_NC * _NP, _D), jnp.float32))

  @functools.partial(
      pl.kernel, mesh=_vector_mesh, out_type=out_type,
      scratch_types=[
          pltpu.VMEM((3, _GC, _CHUNK), jnp.int32),
          pltpu.VMEM((3, _GC, _CHUNK), jnp.int32),
          pltpu.VMEM((_CHUNK, _D), jnp.float32),
          pltpu.VMEM((_CHUNK, _D), jnp.float32),
          pltpu.VMEM_SHARED((_NP, _D), jnp.float32),
          pltpu.SemaphoreType.DMA,
          pltpu.SemaphoreType.DMA,
          pltpu.SemaphoreType.DMA,
          pltpu.SemaphoreType.DMA,
      ])
  def k(h_hbm, src_hbm, dst_hbm, *refs):
    if compute_deg:
      (out_hbm, deg_hbm, src_blk, dst_blk, rows0, rows1, acc_sh,
       gsem, ssem0, ssem1, isem) = refs
    else:
      (out_hbm, src_blk, dst_blk, rows0, rows1, acc_sh,
       gsem, ssem0, ssem1, isem) = refs
    core = lax.axis_index("c")
    sub = lax.axis_index("s")

    rbase = sub * _RPT
    nchunks = _RPT // _CHUNK
    rows = (rows0, rows1)
    ssem = (ssem0, ssem1)

    def drain(sem):
      pltpu.make_async_copy(h_hbm.at[pl.ds(0, _CHUNK)], rows0, sem).wait()

    def drain_idx():
      pltpu.make_async_copy(dst_hbm.at[pl.ds(0, _GC)], dst_blk.at[0],
                            isem).wait()

    def zero_acc():
      for j in range(nchunks):
        pltpu.sync_copy(rows0, acc_sh.at[pl.ds(rbase + j * _CHUNK, _CHUNK)])

    def writeout(dst, bounce_pair):
      for j in range(nchunks):
        b = j % 2
        if j >= 2:
          drain(ssem[b])
        pltpu.sync_copy(acc_sh.at[pl.ds(rbase + j * _CHUNK, _CHUNK)],
                        bounce_pair[b])
        pltpu.async_copy(bounce_pair[b],
                         dst.at[pl.ds(core * _NP + rbase + j * _CHUNK,
                                      _CHUNK)],
                         ssem[b])
      drain(ssem[0])
      drain(ssem[1])

    gstart, ngroups = _tile_groups(core, sub)

    _fill_rows(rows0, 0.0)
    zero_acc()
    plsc.subcore_barrier()

    if compute_deg:
      _fill_rows(rows1, 1.0)

      pltpu.sync_copy(dst_hbm.at[pl.ds(gstart * _GC, _GC)], dst_blk.at[0])

      @pl.loop(0, ngroups)
      def _(blk):
        pb = blk % 3

        @pl.when(blk + 1 < ngroups)
        def _():
          pltpu.async_copy(dst_hbm.at[pl.ds((gstart + blk + 1) * _GC, _GC)],
                           dst_blk.at[(blk + 1) % 3], isem)
        for c in range(0, _GC, 2):
          pltpu.async_copy(rows1, acc_sh.at[dst_blk.at[pb].at[c]], ssem0,
                           add=True)
          pltpu.async_copy(rows1, acc_sh.at[dst_blk.at[pb].at[c + 1]], ssem0,
                           add=True)
          drain(ssem0)
          drain(ssem0)

        @pl.when(blk + 1 < ngroups)
        def _():
          drain_idx()
      plsc.subcore_barrier()
      writeout(deg_hbm, (rows0, rows1))
      _fill_rows(rows0, 0.0)
      _fill_rows(rows1, 0.0)
      zero_acc()
      plsc.subcore_barrier()
    else:
      _fill_rows(rows1, 0.0)

    @pl.loop(0, _CHUNK, step=16)
    def _(i):
      dst_blk[2, 0, pl.ds(i, 16)] = jnp.full((16,), 0, jnp.int32) + rbase
    pltpu.async_copy(rows0, acc_sh.at[dst_blk.at[2].at[0]], ssem0, add=True)
    pltpu.async_copy(rows1, acc_sh.at[dst_blk.at[2].at[0]], ssem1, add=True)

    pltpu.sync_copy(src_hbm.at[pl.ds(gstart * _GC, _GC)], src_blk.at[0])
    pltpu.sync_copy(dst_hbm.at[pl.ds(gstart * _GC, _GC)], dst_blk.at[0])

    @pl.loop(0, ngroups)
    def _(blk):
      pb = blk % 3

      @pl.when(blk + 1 < ngroups)
      def _():
        roff2 = (gstart + blk + 1) * _GC
        pltpu.async_copy(src_hbm.at[pl.ds(roff2, _GC)],
                         src_blk.at[(blk + 1) % 3], isem)
        pltpu.async_copy(dst_hbm.at[pl.ds(roff2, _GC)],
                         dst_blk.at[(blk + 1) % 3], isem)
      for c in range(_GC):
        b = c % 2
        drain(ssem[b])
        pltpu.async_copy(h_hbm.at[src_blk.at[pb].at[c]], rows[b], gsem).wait()
        pltpu.async_copy(rows[b], acc_sh.at[dst_blk.at[pb].at[c]], ssem[b],
                         add=True)

      @pl.when(blk + 1 < ngroups)
      def _():
        drain_idx()
        drain_idx()

    drain(ssem0)
    drain(ssem1)
    plsc.subcore_barrier()

    writeout(out_hbm, (rows0, rows1))

  return k


_sc_layer_deg = _make_sc_layer(True)
_sc_layer = _make_sc_layer(False)

_RB = 1000


def _dense_update(x, a0, a1, d0, d1, W_self, W_neigh, b, apply_relu):
  nb = _N // _RB

  def body(x_ref, a0_ref, a1_ref, d0_ref, d1_ref, ws, wn, b_ref, o_ref):
    s = a0_ref[...] + a1_ref[...]
    dcol = d0_ref[...][:, 0:1] + d1_ref[...][:, 0:1]
    hn = s / jnp.maximum(dcol, 1.0)
    out = (jnp.dot(x_ref[...], ws[...], preferred_element_type=jnp.float32,
                   precision=lax.Precision.HIGHEST)
           + jnp.dot(hn, wn[...], preferred_element_type=jnp.float32,
                     precision=lax.Precision.HIGHEST)
           + b_ref[...])
    if apply_relu:
      out = jnp.maximum(out, 0.0)
    o_ref[...] = out

  blk = lambda r: pl.BlockSpec((r, _D), lambda i: (i, 0))
  return pl.pallas_call(
      body,
      grid=(nb,),
      in_specs=[
          blk(_RB), blk(_RB), blk(_RB), blk(_RB), blk(_RB),
          pl.BlockSpec((_D, _D), lambda i: (0, 0)),
          pl.BlockSpec((_D, _D), lambda i: (0, 0)),
          pl.BlockSpec((1, _D), lambda i: (0, 0)),
      ],
      out_specs=blk(_RB),
      out_shape=jax.ShapeDtypeStruct((_N, _D), jnp.float32),
  )(x, a0, a1, d0, d1, W_self, W_neigh, b.reshape(1, _D))


def _halves(arr):
  return arr[:_N], arr[_NP:_NP + _N]


def kernel(x, edge_index, W_self1, W_neigh1, b1, W_self2, W_neigh2, b2):
  src2 = edge_index[0].reshape(_E // _CHUNK, _CHUNK)
  dst2 = edge_index[1].reshape(_E // _CHUNK, _CHUNK)
  acc1, deg = _sc_layer_deg(x, src2, dst2)
  a0, a1 = _halves(acc1)
  d0, d1 = _halves(deg)
  h1 = _dense_update(x, a0, a1, d0, d1, W_self1, W_neigh1, b1, True)
  (acc2,) = _sc_layer(h1, src2, dst2)
  b0, b1_ = _halves(acc2)
  h2 = _dense_update(h1, b0, b1_, d0, d1, W_self2, W_neigh2, b2, False)
  return h2
